# baseline (speedup 1.0000x reference)
import jax
import jax.numpy as jnp
from jax import lax
from jax.experimental import pallas as pl
from jax.experimental.pallas import tpu as pltpu

_SEG_STARTS = (0, 16, 32, 48, 64, 80, 112)
_SEG_ENDS = (16, 32, 48, 64, 80, 112, 128)


def _dotT(a, b):
    return lax.dot_general(a, b, (((1,), (1,)), ((), ())),
                           preferred_element_type=jnp.float32)


def _pad_rows(t, n):
    return jnp.concatenate(
        [t, jnp.zeros((n - t.shape[0], t.shape[1]), t.dtype)], axis=0)


def _encoder_kernel(feats_ref, hand_ref, suit_ref, bid_ref, role_ref,
                    wb1_ref, bb1_ref, wb2_ref, bb2_ref, wf1_ref, bf1_ref,
                    wf2_ref, bf2_ref, out_ref, win_s, t128_s, wbeh_s, bz_s,
                    wf2_s):
    @pl.when(pl.program_id(0) == 0)
    def _fold():
        wf1 = wf1_ref[...]
        t_hand = _dotT(hand_ref[...], wf1[:, 0:16])
        t_suit = _dotT(0.25 * suit_ref[...], wf1[:, 16:32])
        t_bid = _dotT(bid_ref[...], wf1[:, 32:48])
        t_role = _dotT(role_ref[...], wf1[:, 48:64])
        t128 = jnp.concatenate(
            [_pad_rows(t_hand, 16), _pad_rows(t_suit, 16),
             _pad_rows(t_suit, 16), _pad_rows(t_suit, 16),
             _pad_rows(t_suit, 16), _pad_rows(t_bid, 32),
             _pad_rows(t_role, 16)], axis=0)
        t128_s[...] = t128

        wf1_beh = wf1[:, 64:128]
        wbeh_s[...] = lax.dot_general(
            wb2_ref[...], wf1_beh, (((0,), (1,)), ((), ())),
            preferred_element_type=jnp.float32)
        bz_s[...] = bf1_ref[...] + _dotT(bb2_ref[...], wf1_beh)
        wf2_s[...] = wf2_ref[...]

        row16 = lax.broadcasted_iota(jnp.int32, (16, 128), 0)
        lane = lax.broadcasted_iota(jnp.int32, (16, 128), 1)
        smat = jnp.zeros((16, 128), jnp.float32)
        for c, (s, e) in enumerate(zip(_SEG_STARTS, _SEG_ENDS)):
            smat = smat + jnp.where(
                (row16 == c) & (lane >= s) & (lane < e), 1.0, 0.0)
        wb1t = jnp.concatenate(
            [jnp.zeros((7, 64), jnp.float32), wb1_ref[...].T,
             jnp.zeros((1, 64), jnp.float32)], axis=0)
        win_s[...] = jnp.concatenate(
            [smat, wb1t], axis=1)

    f = feats_ref[...]

    o = lax.dot_general(f, win_s[0:15, :], (((1,), (0,)), ((), ())),
                        preferred_element_type=jnp.float32)
    bc = o[:, 0:128]
    h = jnp.maximum(o[:, 128:192] + bb1_ref[...], 0.0)

    lane1 = lax.broadcasted_iota(jnp.int32, (1, 128), 1)
    seg_start = jnp.zeros((1, 128), jnp.int32)
    for s, e in zip(_SEG_STARTS, _SEG_ENDS):
        seg_start = jnp.where((lane1 >= s) & (lane1 < e), s, seg_start)
    cst = (lane1 - seg_start).astype(jnp.float32)

    mh = (bc == cst).astype(jnp.float32)

    z = (jnp.dot(mh, t128_s[...], preferred_element_type=jnp.float32)
         + jnp.dot(h, wbeh_s[...], preferred_element_type=jnp.float32)
         + bz_s[...])
    g = jnp.maximum(z, 0.0)
    out_ref[...] = _dotT(g, wf2_s[...]) + bf2_ref[...]


@jax.jit
def kernel(player_features, hand_tab, suit_tab, bid_tab, role_tab,
           Wb1, bb1, Wb2, bb2, Wf1, bf1, Wf2, bf2):
    B, P, D = player_features.shape[0], player_features.shape[1], Wf1.shape[0]
    N = B * P
    feats = player_features.reshape(N, 15)

    R = 4096
    grid = (N // R,)

    def full(shape):
        return pl.BlockSpec(shape, lambda i: (0,) * len(shape))

    out = pl.pallas_call(
        _encoder_kernel,
        grid=grid,
        in_specs=[
            pl.BlockSpec((R, 15), lambda i: (i, 0)),
            full((9, 16)), full((9, 16)), full((29, 16)), full((4, 16)),
            full((64, 8)), full((1, 64)), full((64, 64)), full((1, 64)),
            full((128, 128)), full((1, 128)), full((128, 128)),
            full((1, 128)),
        ],
        out_specs=pl.BlockSpec((R, 128), lambda i: (i, 0)),
        out_shape=jax.ShapeDtypeStruct((N, D), jnp.float32),
        scratch_shapes=[
            pltpu.VMEM((16, 192), jnp.float32),
            pltpu.VMEM((128, 128), jnp.float32),
            pltpu.VMEM((64, 128), jnp.float32),
            pltpu.VMEM((1, 128), jnp.float32),
            pltpu.VMEM((128, 128), jnp.float32),
        ],
    )(feats, hand_tab, suit_tab, bid_tab, role_tab, Wb1,
      bb1.reshape(1, 64), Wb2, bb2.reshape(1, 64), Wf1, bf1.reshape(1, 128),
      Wf2, bf2.reshape(1, 128))
    return out.reshape(B, P, D)


# speedup vs baseline: 1.1240x; 1.1240x over previous
import jax
import jax.numpy as jnp
from jax import lax
from jax.experimental import pallas as pl
from jax.experimental.pallas import tpu as pltpu

_SEG_STARTS = (0, 16, 32, 48, 64, 80, 112)
_SEG_ENDS = (16, 32, 48, 64, 80, 112, 128)


def _dotT(a, b):
    return lax.dot_general(a, b, (((1,), (1,)), ((), ())),
                           preferred_element_type=jnp.float32)


def _pad_rows(t, n):
    return jnp.concatenate(
        [t, jnp.zeros((n - t.shape[0], t.shape[1]), t.dtype)], axis=0)


def _encoder_kernel(feats_ref, hand_ref, suit_ref, bid_ref, role_ref,
                    wb1_ref, bb1_ref, wb2_ref, bb2_ref, wf1_ref, bf1_ref,
                    wf2_ref, bf2_ref, out_ref, win_s, t128_s, wbeh_s, bz_s,
                    wf2_s):
    @pl.when(pl.program_id(0) == 0)
    def _fold():
        wf1 = wf1_ref[...]
        t_hand = _dotT(hand_ref[...], wf1[:, 0:16])
        t_suit = _dotT(0.25 * suit_ref[...], wf1[:, 16:32])
        t_bid = _dotT(bid_ref[...], wf1[:, 32:48])
        t_role = _dotT(role_ref[...], wf1[:, 48:64])
        t128 = jnp.concatenate(
            [_pad_rows(t_hand, 16), _pad_rows(t_suit, 16),
             _pad_rows(t_suit, 16), _pad_rows(t_suit, 16),
             _pad_rows(t_suit, 16), _pad_rows(t_bid, 32),
             _pad_rows(t_role, 16)], axis=0)
        t128_s[...] = t128

        wf1_beh = wf1[:, 64:128]
        wbeh_s[...] = lax.dot_general(
            wb2_ref[...], wf1_beh, (((0,), (1,)), ((), ())),
            preferred_element_type=jnp.float32)
        bz_s[...] = bf1_ref[...] + _dotT(bb2_ref[...], wf1_beh)
        wf2_s[...] = wf2_ref[...]

        row16 = lax.broadcasted_iota(jnp.int32, (16, 128), 0)
        lane = lax.broadcasted_iota(jnp.int32, (16, 128), 1)
        smat = jnp.zeros((16, 128), jnp.float32)
        for c, (s, e) in enumerate(zip(_SEG_STARTS, _SEG_ENDS)):
            smat = smat + jnp.where(
                (row16 == c) & (lane >= s) & (lane < e), 1.0, 0.0)
        wb1t = jnp.concatenate(
            [jnp.zeros((7, 64), jnp.float32), wb1_ref[...].T,
             jnp.zeros((1, 64), jnp.float32)], axis=0)
        win_s[...] = jnp.concatenate(
            [smat, wb1t], axis=1)

    f = feats_ref[...]

    h = jnp.maximum(_dotT(f[:, 7:15], wb1_ref[...]) + bb1_ref[...], 0.0)
    bc = jnp.dot(f[:, 0:8], win_s[0:8, 0:128],
                 preferred_element_type=jnp.float32)

    lane1 = lax.broadcasted_iota(jnp.int32, (1, 128), 1)
    seg_start = jnp.zeros((1, 128), jnp.int32)
    for s, e in zip(_SEG_STARTS, _SEG_ENDS):
        seg_start = jnp.where((lane1 >= s) & (lane1 < e), s, seg_start)
    cst = (lane1 - seg_start).astype(jnp.float32)

    mh = (bc == cst).astype(jnp.float32)

    z = (jnp.dot(mh, t128_s[...], preferred_element_type=jnp.float32)
         + jnp.dot(h, wbeh_s[...], preferred_element_type=jnp.float32)
         + bz_s[...])
    g = jnp.maximum(z, 0.0)
    out_ref[...] = _dotT(g, wf2_s[...]) + bf2_ref[...]


@jax.jit
def kernel(player_features, hand_tab, suit_tab, bid_tab, role_tab,
           Wb1, bb1, Wb2, bb2, Wf1, bf1, Wf2, bf2):
    B, P, D = player_features.shape[0], player_features.shape[1], Wf1.shape[0]
    N = B * P
    feats = player_features.reshape(N, 15)

    R = 16384
    grid = (N // R,)

    def full(shape):
        return pl.BlockSpec(shape, lambda i: (0,) * len(shape))

    out = pl.pallas_call(
        _encoder_kernel,
        grid=grid,
        in_specs=[
            pl.BlockSpec((R, 15), lambda i: (i, 0)),
            full((9, 16)), full((9, 16)), full((29, 16)), full((4, 16)),
            full((64, 8)), full((1, 64)), full((64, 64)), full((1, 64)),
            full((128, 128)), full((1, 128)), full((128, 128)),
            full((1, 128)),
        ],
        out_specs=pl.BlockSpec((R, 128), lambda i: (i, 0)),
        out_shape=jax.ShapeDtypeStruct((N, D), jnp.float32),
        scratch_shapes=[
            pltpu.VMEM((16, 192), jnp.float32),
            pltpu.VMEM((128, 128), jnp.float32),
            pltpu.VMEM((64, 128), jnp.float32),
            pltpu.VMEM((1, 128), jnp.float32),
            pltpu.VMEM((128, 128), jnp.float32),
        ],
    )(feats, hand_tab, suit_tab, bid_tab, role_tab, Wb1,
      bb1.reshape(1, 64), Wb2, bb2.reshape(1, 64), Wf1, bf1.reshape(1, 128),
      Wf2, bf2.reshape(1, 128))
    return out.reshape(B, P, D)
